# baseline (speedup 1.0000x reference)
import jax
import jax.numpy as jnp
from jax import lax
from jax.experimental import pallas as pl
from jax.experimental.pallas import tpu as pltpu

_PC = 512
_CHUNKS = 8
_P = _PC * _CHUNKS


def _gather2_kernel(fea_ref, table_ref, out_ref):
    num_hi, wide = table_ref.shape
    d = wide // 4
    num_location = num_hi * 4
    nf = fea_ref.shape[1] // 2

    table = table_ref[...]
    th = table.astype(jnp.bfloat16)
    tl = (table - th.astype(jnp.float32)).astype(jnp.bfloat16)

    for c in range(_CHUNKS):
        iota = lax.broadcasted_iota(jnp.int32, (_PC, num_hi), 1)
        sels = []
        for e in range(2):
            tgt = fea_ref[pl.ds(c * _PC, _PC), nf * e : nf * e + 1]
            tgt = jnp.clip(tgt, 0, num_location - 1)
            hi = tgt >> 2
            lo = tgt & 3
            onehot = jnp.where(iota == hi, 1.0, 0.0).astype(jnp.bfloat16)
            partial = jnp.dot(
                onehot, th, preferred_element_type=jnp.float32
            ) + jnp.dot(
                onehot, tl, preferred_element_type=jnp.float32
            )
            a = partial[:, : 2 * d]
            b = partial[:, 2 * d :]
            sel1 = jnp.where(lo >= 2, b, a)
            rolled = pltpu.roll(sel1, d, axis=1)
            sel2 = jnp.where((lo & 1) == 1, rolled, sel1)
            sels.append(sel2)

        lane = lax.broadcasted_iota(jnp.int32, (_PC, 2 * d), 1)
        out_ref[pl.ds(c * _PC, _PC), :] = jnp.where(
            lane < d, sels[0], pltpu.roll(sels[1], d, axis=1)
        )


def _gather2_call(fea2, table4, num_blocks, p_rows):
    n_packed = fea2.shape[0]
    nf2 = fea2.shape[1]
    num_hi, wide = table4.shape

    return pl.pallas_call(
        _gather2_kernel,
        out_shape=jax.ShapeDtypeStruct((n_packed, wide // 2), jnp.float32),
        grid=(num_blocks,),
        in_specs=[
            pl.BlockSpec((p_rows, nf2), lambda i: (i, 0)),
            pl.BlockSpec((num_hi, wide), lambda i: (0, 0)),
        ],
        out_specs=pl.BlockSpec((p_rows, wide // 2), lambda i: (i, 0)),
        compiler_params=pltpu.CompilerParams(
            dimension_semantics=("arbitrary",),
            vmem_limit_bytes=64 * 1024 * 1024,
        ),
    )(fea2, table4)


def kernel(user_fea, embedding_location):
    n, nf = user_fea.shape
    num_location, d = embedding_location.shape
    assert num_location % 4 == 0 and d % 2 == 0

    rows_per_block = 2 * _P
    n_pad = ((n + rows_per_block - 1) // rows_per_block) * rows_per_block
    if n_pad != n:
        user_fea = jnp.pad(user_fea, ((0, n_pad - n), (0, 0)))
    fea2 = user_fea.reshape(n_pad // 2, 2 * nf).astype(jnp.int32)

    table4 = embedding_location.reshape(num_location // 4, 4 * d)

    nb_total = (n_pad // 2) // _P
    out = _gather2_call(fea2, table4, nb_total, _P)
    return out.reshape(n_pad, d)[:n]


# speedup vs baseline: 1.1579x; 1.1579x over previous
import jax
import jax.numpy as jnp
from jax import lax
from jax.experimental import pallas as pl
from jax.experimental.pallas import tpu as pltpu

_PC = 512
_CHUNKS = 8
_P = _PC * _CHUNKS


def _gather2_kernel(idx_ref, table_ref, out_ref):
    num_hi, wide = table_ref.shape
    d = wide // 4

    table = table_ref[...]
    for c in range(_CHUNKS):
        iota = lax.broadcasted_iota(jnp.int32, (_PC, num_hi), 1)
        sels = []
        for e in range(2):
            tgt = idx_ref[pl.ds(c * _PC, _PC), e : e + 1]
            hi = tgt >> 2
            lo = tgt & 3
            onehot = jnp.where(iota == hi, 1.0, 0.0)
            partial = jnp.dot(
                onehot, table, preferred_element_type=jnp.float32
            )
            a = partial[:, : 2 * d]
            b = partial[:, 2 * d :]
            sel1 = jnp.where(lo >= 2, b, a)
            rolled = pltpu.roll(sel1, d, axis=1)
            sel2 = jnp.where((lo & 1) == e, sel1, rolled)
            sels.append(sel2)

        lane = lax.broadcasted_iota(jnp.int32, (_PC, 2 * d), 1)
        out_ref[pl.ds(c * _PC, _PC), :] = jnp.where(
            lane < d, sels[0], sels[1]
        )


def _gather2_call(idx2, table4, num_blocks, p_rows):
    n_packed = idx2.shape[0]
    num_hi, wide = table4.shape

    return pl.pallas_call(
        _gather2_kernel,
        out_shape=jax.ShapeDtypeStruct((n_packed, wide // 2), jnp.float32),
        grid=(num_blocks,),
        in_specs=[
            pl.BlockSpec((p_rows, 2), lambda i: (i, 0)),
            pl.BlockSpec((num_hi, wide), lambda i: (0, 0)),
        ],
        out_specs=pl.BlockSpec((p_rows, wide // 2), lambda i: (i, 0)),
        compiler_params=pltpu.CompilerParams(
            dimension_semantics=("arbitrary",),
            vmem_limit_bytes=64 * 1024 * 1024,
        ),
    )(idx2, table4)


def kernel(user_fea, embedding_location):
    n = user_fea.shape[0]
    num_location, d = embedding_location.shape
    assert num_location % 4 == 0 and d % 2 == 0

    idx = jnp.clip(user_fea[:, 0].astype(jnp.int32), 0, num_location - 1)

    rows_per_block = 2 * _P
    n_pad = ((n + rows_per_block - 1) // rows_per_block) * rows_per_block
    if n_pad != n:
        idx = jnp.pad(idx, (0, n_pad - n))
    idx2 = idx.reshape(n_pad // 2, 2)

    table4 = embedding_location.reshape(num_location // 4, 4 * d)

    nb_total = (n_pad // 2) // _P
    out = _gather2_call(idx2, table4, nb_total, _P)
    return out.reshape(n_pad, d)[:n]


# speedup vs baseline: 1.4575x; 1.2587x over previous
import jax
import jax.numpy as jnp
from jax import lax
from jax.experimental import pallas as pl
from jax.experimental.pallas import tpu as pltpu

_PC = 8192
_CHUNKS = 1
_P = _PC * _CHUNKS


def _gather2_kernel(idx_ref, table_ref, out_ref):
    num_hi, wide = table_ref.shape
    d = wide // 4

    table = table_ref[...]
    for c in range(_CHUNKS):
        iota = lax.broadcasted_iota(jnp.int32, (_PC, num_hi), 1)
        sels = []
        for e in range(2):
            tgt = idx_ref[pl.ds(c * _PC, _PC), e : e + 1]
            hi = tgt >> 2
            lo = tgt & 3
            onehot = jnp.where(iota == hi, 1.0, 0.0)
            partial = jnp.dot(
                onehot, table, preferred_element_type=jnp.float32
            )
            a = partial[:, : 2 * d]
            b = partial[:, 2 * d :]
            sel1 = jnp.where(lo >= 2, b, a)
            rolled = pltpu.roll(sel1, d, axis=1)
            sel2 = jnp.where((lo & 1) == e, sel1, rolled)
            sels.append(sel2)

        lane = lax.broadcasted_iota(jnp.int32, (_PC, 2 * d), 1)
        out_ref[pl.ds(c * _PC, _PC), :] = jnp.where(
            lane < d, sels[0], sels[1]
        )


def _gather2_call(idx2, table4, num_blocks, p_rows):
    n_packed = idx2.shape[0]
    num_hi, wide = table4.shape

    return pl.pallas_call(
        _gather2_kernel,
        out_shape=jax.ShapeDtypeStruct((n_packed, wide // 2), jnp.float32),
        grid=(num_blocks,),
        in_specs=[
            pl.BlockSpec((p_rows, 2), lambda i: (i, 0)),
            pl.BlockSpec((num_hi, wide), lambda i: (0, 0)),
        ],
        out_specs=pl.BlockSpec((p_rows, wide // 2), lambda i: (i, 0)),
        compiler_params=pltpu.CompilerParams(
            dimension_semantics=("arbitrary",),
            vmem_limit_bytes=64 * 1024 * 1024,
        ),
    )(idx2, table4)


def kernel(user_fea, embedding_location):
    n = user_fea.shape[0]
    num_location, d = embedding_location.shape
    assert num_location % 4 == 0 and d % 2 == 0

    idx = jnp.clip(user_fea[:, 0].astype(jnp.int32), 0, num_location - 1)

    rows_per_block = 2 * _P
    n_pad = ((n + rows_per_block - 1) // rows_per_block) * rows_per_block
    if n_pad != n:
        idx = jnp.pad(idx, (0, n_pad - n))
    idx2 = idx.reshape(n_pad // 2, 2)

    table4 = embedding_location.reshape(num_location // 4, 4 * d)

    nb_total = (n_pad // 2) // _P
    out = _gather2_call(idx2, table4, nb_total, _P)
    return out.reshape(n_pad, d)[:n]
